# baseline (speedup 1.0000x reference)
import functools

import jax
import jax.numpy as jnp
from jax import lax
from jax.experimental import pallas as pl
from jax.experimental.pallas import tpu as pltpu
from jax.experimental.pallas import tpu_sc as plsc

_TOP_K = 8
_LANES = 16


_REC = 128


def _logits_body(x_ref, w_ref, out_ref):
    out_ref[:, : w_ref.shape[0]] = lax.dot_general(
        x_ref[...], w_ref[...], (((1,), (1,)), ((), ())),
        preferred_element_type=jnp.float32)


def _matmul_logits(x, W, off, Tc, BT=512):
    T, D = x.shape
    E = W.shape[0]
    return pl.pallas_call(
        _logits_body,
        grid=(Tc // BT,),
        in_specs=[
            pl.BlockSpec((BT, D), lambda i: (i + off, 0)),
            pl.BlockSpec((E, D), lambda i: (0, 0)),
        ],
        out_specs=pl.BlockSpec((BT, _REC), lambda i: (i, 0)),
        out_shape=jax.ShapeDtypeStruct((Tc, _REC), jnp.float32),
    )(x, W)


@functools.cache
def _make_sc_topk(T, E):
    info = plsc.get_sparse_core_info()
    NC, NS, L = info.num_cores, info.num_subcores, info.num_lanes
    NW = NC * NS
    TW = T // NW
    NCH = E // L
    mesh = plsc.VectorSubcoreMesh(core_axis_name="c", subcore_axis_name="s")

    @functools.partial(
        pl.kernel,
        mesh=mesh,
        compiler_params=pltpu.CompilerParams(needs_layout_passes=False),
        out_type=[
            jax.ShapeDtypeStruct((T * _TOP_K,), jnp.float32),
            jax.ShapeDtypeStruct((T * _TOP_K,), jnp.int32),
        ],
        scratch_types=[
            pltpu.VMEM((TW * _REC,), jnp.float32),
            pltpu.VMEM((TW * _TOP_K,), jnp.float32),
            pltpu.VMEM((TW * _TOP_K,), jnp.int32),
        ],
    )
    def sc_topk(lg_hbm, probs_hbm, idx_hbm, vals_v, pbuf_v, ibuf_v):
        wid = lax.axis_index("s") * NC + lax.axis_index("c")
        base = wid * TW
        pltpu.sync_copy(lg_hbm.at[pl.ds(base * _REC, TW * _REC)], vals_v)

        iota = lax.broadcasted_iota(jnp.int32, (L,), 0)
        idx_c = [iota + L * c for c in range(NCH)]
        lane_lt_k = iota < _TOP_K

        def bitonic_top(a, b, descending):
            (ak, ai), (bk, bi) = a, b
            c = ak >= bk
            mk = jnp.where(c, ak, bk)
            mi = jnp.where(c, ai, bi)
            return plsc.sort_key_val(mk, mi, descending=descending)

        def body(t, carry):
            off = t * _REC
            srt = [
                plsc.sort_key_val(
                    vals_v[pl.ds(off + L * c, L)], idx_c[c],
                    descending=(c % 2 == 0))
                for c in range(NCH)
            ]
            s01 = bitonic_top(srt[0], srt[1], True)
            s23 = bitonic_top(srt[2], srt[3], False)
            sk, si = bitonic_top(s01, s23, True)
            ex = jnp.where(lane_lt_k, jnp.exp(sk), 0.0)
            tot = jnp.sum(ex)
            tgt = iota + t * _TOP_K
            plsc.store_scatter(pbuf_v, [tgt], ex / tot, mask=lane_lt_k)
            plsc.store_scatter(ibuf_v, [tgt], si, mask=lane_lt_k)
            return carry

        lax.fori_loop(0, TW, body, 0, unroll=4)
        pltpu.sync_copy(pbuf_v, probs_hbm.at[pl.ds(base * _TOP_K, TW * _TOP_K)])
        pltpu.sync_copy(ibuf_v, idx_hbm.at[pl.ds(base * _TOP_K, TW * _TOP_K)])

    return sc_topk


def kernel(input, W):
    T, D = input.shape
    E = W.shape[0]
    sizes = [8192, 8192, 8192, 8192]
    BT = 512
    off = 0
    ps, is_ = [], []
    for Tc in sizes:
        li = _matmul_logits(input, W, off // BT, Tc)
        p16, i16 = _make_sc_topk(Tc, E)(li.reshape(-1))
        ps.append(p16.reshape(Tc, _TOP_K))
        is_.append(i16.reshape(Tc, _TOP_K))
        off += Tc
    return jnp.concatenate(ps, 0), jnp.concatenate(is_, 0)


# speedup vs baseline: 1.0306x; 1.0306x over previous
import functools

import jax
import jax.numpy as jnp
from jax import lax
from jax.experimental import pallas as pl
from jax.experimental.pallas import tpu as pltpu
from jax.experimental.pallas import tpu_sc as plsc

_TOP_K = 8
_LANES = 16


_REC = 128


def _logits_body(x_ref, w_ref, out_ref):
    out_ref[:, : w_ref.shape[0]] = lax.dot_general(
        x_ref[...], w_ref[...], (((1,), (1,)), ((), ())),
        preferred_element_type=jnp.float32)


def _matmul_logits(x, W, off, Tc, BT=512):
    T, D = x.shape
    E = W.shape[0]
    return pl.pallas_call(
        _logits_body,
        grid=(Tc // BT,),
        in_specs=[
            pl.BlockSpec((BT, D), lambda i: (i + off, 0)),
            pl.BlockSpec((E, D), lambda i: (0, 0)),
        ],
        out_specs=pl.BlockSpec((BT, _REC), lambda i: (i, 0)),
        out_shape=jax.ShapeDtypeStruct((Tc, _REC), jnp.float32),
    )(x, W)


@functools.cache
def _make_sc_topk(T, E):
    info = plsc.get_sparse_core_info()
    NC, NS, L = info.num_cores, info.num_subcores, info.num_lanes
    NW = NC * NS
    TW = T // NW
    NCH = E // L
    mesh = plsc.VectorSubcoreMesh(core_axis_name="c", subcore_axis_name="s")

    @functools.partial(
        pl.kernel,
        mesh=mesh,
        compiler_params=pltpu.CompilerParams(needs_layout_passes=False),
        out_type=[
            jax.ShapeDtypeStruct((T * _LANES,), jnp.float32),
            jax.ShapeDtypeStruct((T * _LANES,), jnp.int32),
        ],
        scratch_types=[
            pltpu.VMEM((TW * _REC,), jnp.float32),
            pltpu.VMEM((TW * _LANES,), jnp.float32),
            pltpu.VMEM((TW * _LANES,), jnp.int32),
        ],
    )
    def sc_topk(lg_hbm, probs_hbm, idx_hbm, vals_v, pbuf_v, ibuf_v):
        wid = lax.axis_index("s") * NC + lax.axis_index("c")
        base = wid * TW
        pltpu.sync_copy(lg_hbm.at[pl.ds(base * _REC, TW * _REC)], vals_v)

        iota = lax.broadcasted_iota(jnp.int32, (L,), 0)
        idx_c = [iota + L * c for c in range(NCH)]
        lane_lt_k = iota < _TOP_K

        def bitonic_top(a, b, descending):
            (ak, ai), (bk, bi) = a, b
            c = ak >= bk
            mk = jnp.where(c, ak, bk)
            mi = jnp.where(c, ai, bi)
            return plsc.sort_key_val(mk, mi, descending=descending)

        def body(t, carry):
            off = t * _REC
            srt = [
                plsc.sort_key_val(
                    vals_v[pl.ds(off + L * c, L)], idx_c[c],
                    descending=(c % 2 == 0))
                for c in range(NCH)
            ]
            s01 = bitonic_top(srt[0], srt[1], True)
            s23 = bitonic_top(srt[2], srt[3], False)
            sk, si = bitonic_top(s01, s23, True)
            ex = jnp.where(lane_lt_k, jnp.exp(sk), 0.0)
            tot = jnp.sum(ex)
            pbuf_v[pl.ds(t * _LANES, _LANES)] = ex / tot
            ibuf_v[pl.ds(t * _LANES, _LANES)] = si
            return carry

        lax.fori_loop(0, TW, body, 0, unroll=4)
        pltpu.sync_copy(pbuf_v, probs_hbm.at[pl.ds(base * _LANES, TW * _LANES)])
        pltpu.sync_copy(ibuf_v, idx_hbm.at[pl.ds(base * _LANES, TW * _LANES)])

    return sc_topk


def kernel(input, W):
    T, D = input.shape
    E = W.shape[0]
    sizes = [8192, 8192, 8192, 8192]
    BT = 512
    off = 0
    ps, is_ = [], []
    for Tc in sizes:
        li = _matmul_logits(input, W, off // BT, Tc)
        p16, i16 = _make_sc_topk(Tc, E)(li.reshape(-1))
        ps.append(p16.reshape(Tc, _LANES)[:, :_TOP_K])
        is_.append(i16.reshape(Tc, _LANES)[:, :_TOP_K])
        off += Tc
    return jnp.concatenate(ps, 0), jnp.concatenate(is_, 0)
